# speedup vs baseline: 2.4826x; 2.4826x over previous
import functools

import jax
import jax.numpy as jnp
from jax import lax
from jax.experimental import pallas as pl
from jax.experimental.pallas import tpu as pltpu
from jax.experimental.pallas import tpu_sc as plsc

_N = 10000
_E = 320000
_D = 128
_H = 128

_NC = 2
_NS = 16
_NW = _NC * _NS
_PER_W = _E // _NW
_C = 80
_CHUNKS = _PER_W // _C

_BE = 1280
_LN_EPS = 1e-5


def _proj_body(nf_ref, wst_ref, wdt_ref, s_ref, d_ref):
    nf = nf_ref[...]
    s_ref[...] = jnp.dot(nf, wst_ref[...], preferred_element_type=jnp.float32)
    d_ref[...] = jnp.dot(nf, wdt_ref[...], preferred_element_type=jnp.float32)


def _proj_call(nfeat, wst, wdt):
    return pl.pallas_call(
        _proj_body,
        out_shape=(
            jax.ShapeDtypeStruct((_N, _H), jnp.float32),
            jax.ShapeDtypeStruct((_N, _H), jnp.float32),
        ),
    )(nfeat, wst, wdt)


def _gather_body(s_hbm, d_hbm, src_hbm, dst_hbm, out_hbm,
                 idx_s, idx_d, buf_s, buf_d, sem_s, sem_d):
    wid = lax.axis_index("s") * _NC + lax.axis_index("c")
    base = wid * _PER_W

    def chunk(i, carry):
        off = pl.multiple_of(base + i * _C, 8)
        pltpu.sync_copy(src_hbm.at[pl.ds(off, _C)], idx_s)
        pltpu.sync_copy(dst_hbm.at[pl.ds(off, _C)], idx_d)
        cp_s = pltpu.async_copy(s_hbm.at[idx_s], buf_s, sem_s)
        cp_d = pltpu.async_copy(d_hbm.at[idx_d], buf_d, sem_d)
        cp_s.wait()
        cp_d.wait()

        def row(r, c2):
            b = r * 2
            for rr in range(2):
                for k in range(_H // 16):
                    sl = pl.ds(k * 16, 16)
                    buf_s[b + rr, sl] = buf_s[b + rr, sl] + buf_d[b + rr, sl]
            return c2

        lax.fori_loop(0, _C // 2, row, 0)
        pltpu.sync_copy(buf_s, out_hbm.at[pl.ds(off, _C)])
        return carry

    lax.fori_loop(0, _CHUNKS, chunk, 0)


def _gather_call(mlp_s, mlp_d, src, dst):
    mesh = plsc.VectorSubcoreMesh(
        core_axis_name="c", subcore_axis_name="s",
        num_cores=_NC, num_subcores=_NS)
    fn = pl.kernel(
        _gather_body,
        out_type=jax.ShapeDtypeStruct((_E, _H), jnp.float32),
        mesh=mesh,
        scratch_types=[
            pltpu.VMEM((_C,), jnp.int32),
            pltpu.VMEM((_C,), jnp.int32),
            pltpu.VMEM((_C, _H), jnp.float32),
            pltpu.VMEM((_C, _H), jnp.float32),
            pltpu.SemaphoreType.DMA,
            pltpu.SemaphoreType.DMA,
        ],
    )
    return fn(mlp_s, mlp_d, src, dst)


def _edge_body(e_ref, g_ref, wet_ref, wot_ref, b1_ref, bo_ref,
               lng_ref, lnb_ref, out_ref):
    e = e_ref[...]
    h = jnp.dot(e, wet_ref[...], preferred_element_type=jnp.float32)
    h = h + g_ref[...] + b1_ref[...]
    h = h / (1.0 + jnp.exp(-h))
    o = jnp.dot(h, wot_ref[...], preferred_element_type=jnp.float32)
    o = o + bo_ref[...]
    m = jnp.mean(o, axis=-1, keepdims=True)
    c = o - m
    v = jnp.mean(c * c, axis=-1, keepdims=True)
    o = c / jnp.sqrt(v + _LN_EPS) * lng_ref[...] + lnb_ref[...]
    out_ref[...] = o + e


def _edge_call(efeat, g, wet, wot, b1, bo, lng, lnb):
    grid = (_E // _BE,)
    blk = pl.BlockSpec((_BE, _D), lambda i: (i, 0))
    full = pl.BlockSpec((_D, _H), lambda i: (0, 0))
    vec = pl.BlockSpec((1, _H), lambda i: (0, 0))
    return pl.pallas_call(
        _edge_body,
        grid=grid,
        in_specs=[blk, blk, full, full, vec, vec, vec, vec],
        out_specs=blk,
        out_shape=jax.ShapeDtypeStruct((_E, _D), jnp.float32),
    )(efeat, g, wet, wot, b1, bo, lng, lnb)


def kernel(efeat, nfeat, src, dst, W_e, W_s, W_d, b1, W_o, b_o, ln_g, ln_b):
    mlp_s, mlp_d = _proj_call(nfeat, W_s.T, W_d.T)
    g = _gather_call(mlp_s, mlp_d, src, dst)
    out = _edge_call(
        efeat, g, W_e.T, W_o.T,
        b1.reshape(1, _H), b_o.reshape(1, _D),
        ln_g.reshape(1, _D), ln_b.reshape(1, _D))
    return (out, nfeat)


# speedup vs baseline: 3.7066x; 1.4930x over previous
import functools

import jax
import jax.numpy as jnp
from jax import lax
from jax.experimental import pallas as pl
from jax.experimental.pallas import tpu as pltpu
from jax.experimental.pallas import tpu_sc as plsc

_N = 10000
_E = 320000
_D = 128
_H = 128

_NC = 2
_NS = 16
_NW = _NC * _NS
_PER_W = _E // _NW
_C = 80
_CHUNKS = _PER_W // _C

_BE = 1280
_LN_EPS = 1e-5


def _proj_body(nf_ref, wst_ref, wdt_ref, s_ref, d_ref):
    nf = nf_ref[...]
    s_ref[...] = jnp.dot(nf, wst_ref[...], preferred_element_type=jnp.float32)
    d_ref[...] = jnp.dot(nf, wdt_ref[...], preferred_element_type=jnp.float32)


def _proj_call(nfeat, wst, wdt):
    return pl.pallas_call(
        _proj_body,
        out_shape=(
            jax.ShapeDtypeStruct((_N, _H), jnp.float32),
            jax.ShapeDtypeStruct((_N, _H), jnp.float32),
        ),
    )(nfeat, wst, wdt)


def _gather_body(s_hbm, d_hbm, src_hbm, dst_hbm, out_hbm,
                 idx_s, idx_d, buf_s, buf_d, obuf, sem_s, sem_d, sem_o):
    wid = lax.axis_index("s") * _NC + lax.axis_index("c")
    base = wid * _PER_W

    pltpu.sync_copy(src_hbm.at[pl.ds(pl.multiple_of(base, 8), _PER_W)], idx_s)
    pltpu.sync_copy(dst_hbm.at[pl.ds(pl.multiple_of(base, 8), _PER_W)], idx_d)

    def issue(j, slot):
        js = pl.multiple_of(j * _C, 8)
        pltpu.async_copy(s_hbm.at[idx_s.at[pl.ds(js, _C)]], buf_s.at[slot],
                         sem_s.at[slot])
        pltpu.async_copy(d_hbm.at[idx_d.at[pl.ds(js, _C)]], buf_d.at[slot],
                         sem_d.at[slot])

    def process(j, slot, first):
        off = pl.multiple_of(base + j * _C, 8)
        pltpu.make_async_copy(s_hbm.at[idx_s.at[pl.ds(0, _C)]],
                              buf_s.at[slot], sem_s.at[slot]).wait()
        pltpu.make_async_copy(d_hbm.at[idx_d.at[pl.ds(0, _C)]],
                              buf_d.at[slot], sem_d.at[slot]).wait()
        if not first:
            pltpu.make_async_copy(obuf.at[slot],
                                  out_hbm.at[pl.ds(0, _C)], sem_o.at[slot]).wait()

        def row(r, c2):
            b = r * 2
            for rr in range(2):
                for k in range(_H // 16):
                    sl = pl.ds(k * 16, 16)
                    obuf[slot, b + rr, sl] = (
                        buf_s[slot, b + rr, sl] + buf_d[slot, b + rr, sl])
            return c2

        lax.fori_loop(0, _C // 2, row, 0)
        pltpu.async_copy(obuf.at[slot], out_hbm.at[pl.ds(off, _C)],
                         sem_o.at[slot])

    issue(0, 0)
    issue(1, 1)
    process(0, 0, first=True)
    issue(2, 0)
    process(1, 1, first=True)
    issue(3, 1)

    def pair(k, carry):
        j = k * 2
        process(j, 0, first=False)
        issue(j + 2, 0)
        process(j + 1, 1, first=False)
        issue(j + 3, 1)
        return carry

    lax.fori_loop(1, 61, pair, 0)
    process(122, 0, first=False)
    issue(124, 0)
    process(123, 1, first=False)
    process(124, 0, first=False)
    pltpu.make_async_copy(obuf.at[0], out_hbm.at[pl.ds(0, _C)],
                          sem_o.at[0]).wait()
    pltpu.make_async_copy(obuf.at[1], out_hbm.at[pl.ds(0, _C)],
                          sem_o.at[1]).wait()


def _gather_call(mlp_s, mlp_d, src, dst):
    mesh = plsc.VectorSubcoreMesh(
        core_axis_name="c", subcore_axis_name="s",
        num_cores=_NC, num_subcores=_NS)
    fn = pl.kernel(
        _gather_body,
        out_type=jax.ShapeDtypeStruct((_E, _H), jnp.float32),
        mesh=mesh,
        scratch_types=[
            pltpu.VMEM((_PER_W,), jnp.int32),
            pltpu.VMEM((_PER_W,), jnp.int32),
            pltpu.VMEM((2, _C, _H), jnp.float32),
            pltpu.VMEM((2, _C, _H), jnp.float32),
            pltpu.VMEM((2, _C, _H), jnp.float32),
            pltpu.SemaphoreType.DMA((2,)),
            pltpu.SemaphoreType.DMA((2,)),
            pltpu.SemaphoreType.DMA((2,)),
        ],
    )
    return fn(mlp_s, mlp_d, src, dst)


def _edge_body(e_ref, g_ref, wet_ref, wot_ref, b1_ref, bo_ref,
               lng_ref, lnb_ref, out_ref):
    e = e_ref[...]
    h = jnp.dot(e, wet_ref[...], preferred_element_type=jnp.float32)
    h = h + g_ref[...] + b1_ref[...]
    h = h / (1.0 + jnp.exp(-h))
    o = jnp.dot(h, wot_ref[...], preferred_element_type=jnp.float32)
    o = o + bo_ref[...]
    m = jnp.mean(o, axis=-1, keepdims=True)
    c = o - m
    v = jnp.mean(c * c, axis=-1, keepdims=True)
    o = c / jnp.sqrt(v + _LN_EPS) * lng_ref[...] + lnb_ref[...]
    out_ref[...] = o + e


def _edge_call(efeat, g, wet, wot, b1, bo, lng, lnb):
    grid = (_E // _BE,)
    blk = pl.BlockSpec((_BE, _D), lambda i: (i, 0))
    full = pl.BlockSpec((_D, _H), lambda i: (0, 0))
    vec = pl.BlockSpec((1, _H), lambda i: (0, 0))
    return pl.pallas_call(
        _edge_body,
        grid=grid,
        in_specs=[blk, blk, full, full, vec, vec, vec, vec],
        out_specs=blk,
        out_shape=jax.ShapeDtypeStruct((_E, _D), jnp.float32),
    )(efeat, g, wet, wot, b1, bo, lng, lnb)


def kernel(efeat, nfeat, src, dst, W_e, W_s, W_d, b1, W_o, b_o, ln_g, ln_b):
    mlp_s, mlp_d = _proj_call(nfeat, W_s.T, W_d.T)
    g = _gather_call(mlp_s, mlp_d, src, dst)
    out = _edge_call(
        efeat, g, W_e.T, W_o.T,
        b1.reshape(1, _H), b_o.reshape(1, _D),
        ln_g.reshape(1, _D), ln_b.reshape(1, _D))
    return (out, nfeat)
